# baseline (speedup 1.0000x reference)
import functools

import jax
import jax.numpy as jnp
import numpy as np
from jax import lax
from jax.experimental import pallas as pl
from jax.experimental.pallas import tpu as pltpu
from jax.experimental.pallas import tpu_sc as plsc

B = 16384
N_CONT = 13
N_CAT = 26
V = 100001
D = 16
EPS = 1e-5

NC = 2
NS = 16
NW = NC * NS
ROWS_PER_W = B // NW
CHUNK = 128
N_CHUNKS = ROWS_PER_W // CHUNK
IDX_PER_CHUNK = CHUNK * N_CAT
GW = 128
N_G = IDX_PER_CHUNK // GW
VREGS_PER_CHUNK = IDX_PER_CHUNK // 16
PAT_VREGS = 13

CB = 2048
COLS = (N_CONT + N_CAT) * D
CAT_COLS = N_CAT * D
CONT_COLS = N_CONT * D
XF_LANES = 8 * N_CONT
XF_ROWS = (B * N_CONT) // XF_LANES


def _sc_gather(cat_flat, pat, tab_flat):
    mesh = plsc.VectorSubcoreMesh(
        core_axis_name="c", subcore_axis_name="s", num_cores=NC, num_subcores=NS
    )

    @functools.partial(
        pl.kernel,
        out_type=jax.ShapeDtypeStruct((B * N_CAT, D), jnp.float32),
        mesh=mesh,
        scratch_types=[
            pltpu.VMEM((IDX_PER_CHUNK,), jnp.int32),
            pltpu.VMEM((PAT_VREGS * 16,), jnp.int32),
            pltpu.VMEM((IDX_PER_CHUNK, D), jnp.float32),
            pltpu.SemaphoreType.DMA,
        ],
        compiler_params=pltpu.CompilerParams(use_tc_tiling_on_sc=False),
    )
    def body(cat_hbm, pat_hbm, tab_hbm, out_hbm, idx_v, pat_v, rows_v, sem):
        wid = lax.axis_index("s") * NC + lax.axis_index("c")
        base0 = wid * ROWS_PER_W
        pltpu.sync_copy(pat_hbm, pat_v)

        for ci in range(N_CHUNKS):
            base = (base0 + ci * CHUNK) * N_CAT
            pltpu.sync_copy(cat_hbm.at[pl.ds(base, IDX_PER_CHUNK)], idx_v)

            def add_off(v, carry):
                src = pl.multiple_of(v * 16, 16)
                po = pl.multiple_of(lax.rem(v, PAT_VREGS) * 16, 16)
                idx_v[pl.ds(src, 16)] = idx_v[pl.ds(src, 16)] + pat_v[pl.ds(po, 16)]
                return carry

            lax.fori_loop(0, VREGS_PER_CHUNK, add_off, 0)

            def fire(g, carry):
                o = pl.multiple_of(g * GW, GW)
                pltpu.async_copy(
                    tab_hbm.at[idx_v.at[pl.ds(o, GW)]],
                    rows_v.at[pl.ds(o, GW)],
                    sem,
                )
                return carry

            lax.fori_loop(0, N_G, fire, 0)
            pltpu.make_async_copy(
                tab_hbm.at[pl.ds(0, IDX_PER_CHUNK)], rows_v, sem
            ).wait()
            pltpu.sync_copy(rows_v, out_hbm.at[pl.ds(base, IDX_PER_CHUNK)])

    return body(cat_flat, pat, tab_flat)


def _tc_cont_assemble(xf, x, m, e, bias_flat, gamma, beta, xcat):
    def body(xf_ref, x_ref, m_ref, e_ref, bf_ref, g_ref, b_ref, xc_ref, o_ref):
        xfv = xf_ref[...]
        s1 = jnp.sum(xfv, axis=0, keepdims=True)
        s2 = jnp.sum(xfv * xfv, axis=0, keepdims=True)
        mv = m_ref[...]
        fs1 = jnp.dot(s1, mv, preferred_element_type=jnp.float32, precision=lax.Precision.HIGHEST)
        fs2 = jnp.dot(s2, mv, preferred_element_type=jnp.float32, precision=lax.Precision.HIGHEST)
        mean = fs1 * (1.0 / B)
        var = fs2 * (1.0 / B) - mean * mean
        inv = lax.rsqrt(var + EPS) * g_ref[...]
        xn = (x_ref[...] - mean) * inv + b_ref[...]
        cont = (
            jnp.dot(xn, e_ref[...], preferred_element_type=jnp.float32, precision=lax.Precision.HIGHEST)
            + bf_ref[...]
        )
        o_ref[...] = jnp.concatenate([cont, xc_ref[...]], axis=1)

    grid = B // CB
    return pl.pallas_call(
        body,
        grid=(grid,),
        in_specs=[
            pl.BlockSpec((XF_ROWS, XF_LANES), lambda i: (0, 0)),
            pl.BlockSpec((CB, N_CONT), lambda i: (i, 0)),
            pl.BlockSpec((XF_LANES, N_CONT), lambda i: (0, 0)),
            pl.BlockSpec((N_CONT, CONT_COLS), lambda i: (0, 0)),
            pl.BlockSpec((1, CONT_COLS), lambda i: (0, 0)),
            pl.BlockSpec((1, N_CONT), lambda i: (0, 0)),
            pl.BlockSpec((1, N_CONT), lambda i: (0, 0)),
            pl.BlockSpec((CB, CAT_COLS), lambda i: (i, 0)),
        ],
        out_specs=pl.BlockSpec((CB, COLS), lambda i: (i, 0)),
        out_shape=jax.ShapeDtypeStruct((B, COLS), jnp.float32),
        compiler_params=pltpu.CompilerParams(
            dimension_semantics=("arbitrary",),
        ),
    )(xf, x, m, e, bias_flat, gamma, beta, xcat)


def kernel(x, categorical, cont_embed_weight, cont_embed_bias, bn_gamma, bn_beta, cat_tables):
    tab_flat = cat_tables.reshape(N_CAT * V, D)
    cat_flat = categorical.reshape(B * N_CAT)
    pat = jnp.asarray((np.arange(PAT_VREGS * 16) % N_CAT) * V, dtype=jnp.int32)

    xf = x.reshape(XF_ROWS, XF_LANES)
    m = jnp.asarray(
        (np.arange(XF_LANES)[:, None] % N_CONT) == np.arange(N_CONT)[None, :],
        dtype=jnp.float32,
    )
    e = (
        jnp.eye(N_CONT, dtype=jnp.float32)[:, :, None]
        * cont_embed_weight[None, :, :]
    ).reshape(N_CONT, CONT_COLS)
    bias_flat = cont_embed_bias.reshape(1, CONT_COLS)
    gamma = bn_gamma.reshape(1, N_CONT)
    beta = bn_beta.reshape(1, N_CONT)

    rows = _sc_gather(cat_flat, pat, tab_flat)
    xcat = rows.reshape(B, CAT_COLS)

    out = _tc_cont_assemble(xf, x, m, e, bias_flat, gamma, beta, xcat)
    return out.reshape(B, N_CONT + N_CAT, D)


# speedup vs baseline: 2.9327x; 2.9327x over previous
import functools

import jax
import jax.numpy as jnp
import numpy as np
from jax import lax
from jax.experimental import pallas as pl
from jax.experimental.pallas import tpu as pltpu
from jax.experimental.pallas import tpu_sc as plsc

B = 16384
N_CONT = 13
N_CAT = 26
V = 100001
D = 16
EPS = 1e-5

V_PAD = 100352
VCHUNK = V_PAD // 8
RCHUNK = VCHUNK * D // 128
VT = N_CAT * V_PAD

NC = 2
NS = 16
NW = NC * NS
ROWS_PER_W = B // NW
CHUNK = 128
N_CHUNKS = ROWS_PER_W // CHUNK
IDX_PER_CHUNK = CHUNK * N_CAT
GW = 128
N_G = IDX_PER_CHUNK // GW
VREGS_PER_CHUNK = IDX_PER_CHUNK // 16
PAT_VREGS = 13

CB = 2048
COLS = (N_CONT + N_CAT) * D
CAT_COLS = N_CAT * D
CONT_COLS = N_CONT * D
XF_LANES = 8 * N_CONT
XF_ROWS = (B * N_CONT) // XF_LANES


def _tc_repack(tab_t):
    def body(t_ref, o_ref):
        a = t_ref[0]
        b = a.reshape(D, RCHUNK, 8)
        o_ref[...] = jnp.transpose(b, (1, 2, 0)).reshape(RCHUNK, 128)

    return pl.pallas_call(
        body,
        grid=(N_CAT, V_PAD // VCHUNK),
        in_specs=[pl.BlockSpec((1, D, VCHUNK), lambda i, j: (i, 0, j))],
        out_specs=pl.BlockSpec((RCHUNK, 128), lambda i, j: (i * (V_PAD // VCHUNK) + j, 0)),
        out_shape=jax.ShapeDtypeStruct((VT * D // 128, 128), jnp.float32),
        compiler_params=pltpu.CompilerParams(
            dimension_semantics=("arbitrary", "arbitrary"),
        ),
    )(tab_t)


def _sc_gather(cat_flat, pat, tab_flat):
    mesh = plsc.VectorSubcoreMesh(
        core_axis_name="c", subcore_axis_name="s", num_cores=NC, num_subcores=NS
    )

    @functools.partial(
        pl.kernel,
        out_type=jax.ShapeDtypeStruct((B * N_CAT, D), jnp.float32),
        name="sc_embedding_gather",
        mesh=mesh,
        scratch_types=[
            pltpu.VMEM((IDX_PER_CHUNK,), jnp.int32),
            pltpu.VMEM((PAT_VREGS * 16,), jnp.int32),
            pltpu.VMEM((IDX_PER_CHUNK, D), jnp.float32),
            pltpu.SemaphoreType.DMA,
        ],
        compiler_params=pltpu.CompilerParams(use_tc_tiling_on_sc=False),
    )
    def body(cat_hbm, pat_hbm, tab_hbm, out_hbm, idx_v, pat_v, rows_v, sem):
        wid = lax.axis_index("s") * NC + lax.axis_index("c")
        base0 = wid * ROWS_PER_W
        pltpu.sync_copy(pat_hbm, pat_v)

        for ci in range(N_CHUNKS):
            base = (base0 + ci * CHUNK) * N_CAT
            pltpu.sync_copy(cat_hbm.at[pl.ds(base, IDX_PER_CHUNK)], idx_v)

            def add_off(v, carry):
                src = pl.multiple_of(v * 16, 16)
                po = pl.multiple_of(lax.rem(v, PAT_VREGS) * 16, 16)
                idx_v[pl.ds(src, 16)] = idx_v[pl.ds(src, 16)] + pat_v[pl.ds(po, 16)]
                return carry

            lax.fori_loop(0, VREGS_PER_CHUNK, add_off, 0)

            def fire(g, carry):
                o = pl.multiple_of(g * GW, GW)
                pltpu.async_copy(
                    tab_hbm.at[idx_v.at[pl.ds(o, GW)]],
                    rows_v.at[pl.ds(o, GW)],
                    sem,
                )
                return carry

            lax.fori_loop(0, N_G, fire, 0)
            pltpu.make_async_copy(
                tab_hbm.at[pl.ds(0, IDX_PER_CHUNK)], rows_v, sem
            ).wait()
            pltpu.sync_copy(rows_v, out_hbm.at[pl.ds(base, IDX_PER_CHUNK)])

    return body(cat_flat, pat, tab_flat)


def _tc_cont_assemble(xf, x, m, e, bias_flat, gamma, beta, xcat):
    def body(xf_ref, x_ref, m_ref, e_ref, bf_ref, g_ref, b_ref, xc_ref, o_ref):
        xfv = xf_ref[...]
        s1 = jnp.sum(xfv, axis=0, keepdims=True)
        s2 = jnp.sum(xfv * xfv, axis=0, keepdims=True)
        mv = m_ref[...]
        fs1 = jnp.dot(s1, mv, preferred_element_type=jnp.float32, precision=lax.Precision.HIGHEST)
        fs2 = jnp.dot(s2, mv, preferred_element_type=jnp.float32, precision=lax.Precision.HIGHEST)
        mean = fs1 * (1.0 / B)
        var = fs2 * (1.0 / B) - mean * mean
        inv = lax.rsqrt(var + EPS) * g_ref[...]
        xn = (x_ref[...] - mean) * inv + b_ref[...]
        cont = (
            jnp.dot(xn, e_ref[...], preferred_element_type=jnp.float32, precision=lax.Precision.HIGHEST)
            + bf_ref[...]
        )
        o_ref[...] = jnp.concatenate([cont, xc_ref[...]], axis=1)

    grid = B // CB
    return pl.pallas_call(
        body,
        grid=(grid,),
        in_specs=[
            pl.BlockSpec((XF_ROWS, XF_LANES), lambda i: (0, 0)),
            pl.BlockSpec((CB, N_CONT), lambda i: (i, 0)),
            pl.BlockSpec((XF_LANES, N_CONT), lambda i: (0, 0)),
            pl.BlockSpec((N_CONT, CONT_COLS), lambda i: (0, 0)),
            pl.BlockSpec((1, CONT_COLS), lambda i: (0, 0)),
            pl.BlockSpec((1, N_CONT), lambda i: (0, 0)),
            pl.BlockSpec((1, N_CONT), lambda i: (0, 0)),
            pl.BlockSpec((CB, CAT_COLS), lambda i: (i, 0)),
        ],
        out_specs=pl.BlockSpec((CB, COLS), lambda i: (i, 0)),
        out_shape=jax.ShapeDtypeStruct((B, COLS), jnp.float32),
        compiler_params=pltpu.CompilerParams(
            dimension_semantics=("arbitrary",),
        ),
    )(xf, x, m, e, bias_flat, gamma, beta, xcat)


def kernel(x, categorical, cont_embed_weight, cont_embed_bias, bn_gamma, bn_beta, cat_tables):
    tab_t = cat_tables.transpose(0, 2, 1)
    cat_flat = categorical.reshape(B * N_CAT)
    pat = jnp.asarray((np.arange(PAT_VREGS * 16) % N_CAT) * V_PAD, dtype=jnp.int32)

    xf = x.reshape(XF_ROWS, XF_LANES)
    m = jnp.asarray(
        (np.arange(XF_LANES)[:, None] % N_CONT) == np.arange(N_CONT)[None, :],
        dtype=jnp.float32,
    )
    e = (
        jnp.eye(N_CONT, dtype=jnp.float32)[:, :, None]
        * cont_embed_weight[None, :, :]
    ).reshape(N_CONT, CONT_COLS)
    bias_flat = cont_embed_bias.reshape(1, CONT_COLS)
    gamma = bn_gamma.reshape(1, N_CONT)
    beta = bn_beta.reshape(1, N_CONT)

    tab_flat = _tc_repack(tab_t).reshape(VT, D)

    rows = _sc_gather(cat_flat, pat, tab_flat)
    xcat = rows.reshape(B, CAT_COLS)

    out = _tc_cont_assemble(xf, x, m, e, bias_flat, gamma, beta, xcat)
    return out.reshape(B, N_CONT + N_CAT, D)


# speedup vs baseline: 14.7423x; 5.0270x over previous
import functools

import jax
import jax.numpy as jnp
import numpy as np
from jax import lax
from jax.experimental import pallas as pl
from jax.experimental.pallas import tpu as pltpu
from jax.experimental.pallas import tpu_sc as plsc

B = 16384
N_CONT = 13
N_CAT = 26
V = 100001
D = 16
EPS = 1e-5

V_PAD = 100352
PLANE = D * V_PAD

NC = 2
NS = 16
NW = NC * NS
ROWS_PER_W = B // NW
BCHUNK = 512
N_BCHUNK = ROWS_PER_W // BCHUNK

CB = 2048


def _tc_repack(tab_t):
    def body(t_ref, o_ref):
        o_ref[...] = t_ref[0].reshape(D, V_PAD // 128, 128)

    return pl.pallas_call(
        body,
        grid=(N_CAT,),
        in_specs=[pl.BlockSpec((1, D, V_PAD), lambda i: (i, 0, 0))],
        out_specs=pl.BlockSpec((D, V_PAD // 128, 128), lambda i: (i, 0, 0)),
        out_shape=jax.ShapeDtypeStruct((N_CAT * D, V_PAD // 128, 128), jnp.float32),
        compiler_params=pltpu.CompilerParams(
            dimension_semantics=("arbitrary",),
        ),
    )(tab_t)


def _sc_gather(cat_t, tab_lin):
    mesh = plsc.VectorSubcoreMesh(
        core_axis_name="c", subcore_axis_name="s", num_cores=NC, num_subcores=NS
    )

    @functools.partial(
        pl.kernel,
        out_type=jax.ShapeDtypeStruct((N_CAT, D, B), jnp.float32),
        name="sc_embedding_gather",
        mesh=mesh,
        scratch_types=[
            pltpu.VMEM((BCHUNK,), jnp.int32),
            pltpu.VMEM((D, BCHUNK), jnp.float32),
            pltpu.SemaphoreType.DMA,
        ],
        compiler_params=pltpu.CompilerParams(use_tc_tiling_on_sc=False),
    )
    def body(cat_hbm, tab_hbm, out_hbm, idx_v, val_v, sem):
        wid = lax.axis_index("s") * NC + lax.axis_index("c")
        base0 = wid * ROWS_PER_W

        def per_chunk(ci, carry):
            base = base0 + ci * BCHUNK

            def per_t(t, carry2):
                pltpu.sync_copy(cat_hbm.at[t, pl.ds(base, BCHUNK)], idx_v)

                def fire(d, carry3):
                    off = pl.multiple_of((t * D + d) * V_PAD, 128)
                    pltpu.async_copy(
                        tab_hbm.at[pl.ds(off, V_PAD)].at[idx_v],
                        val_v.at[d],
                        sem,
                    )
                    return carry3

                lax.fori_loop(0, D, fire, 0)
                pltpu.make_async_copy(
                    out_hbm.at[0, :, pl.ds(0, BCHUNK)], val_v, sem
                ).wait()
                pltpu.sync_copy(val_v, out_hbm.at[t, :, pl.ds(base, BCHUNK)])
                return carry2

            lax.fori_loop(0, N_CAT, per_t, 0)
            return carry

        lax.fori_loop(0, N_BCHUNK, per_chunk, 0)

    return body(cat_t, tab_lin)


def _tc_cont(x_t, gamma, beta, w_t, b_t):
    def body(x_ref, xc_ref, g_ref, be_ref, w_ref, bb_ref, o_ref):
        xv = x_ref[...]
        mean = jnp.mean(xv, axis=1, keepdims=True)
        var = jnp.mean(xv * xv, axis=1, keepdims=True) - mean * mean
        inv = lax.rsqrt(var + EPS) * g_ref[...]
        xc = (xc_ref[...] - mean) * inv + be_ref[...]
        o_ref[...] = (
            w_ref[...] * xc[:, None, :] + bb_ref[...]
        )

    grid = B // CB
    return pl.pallas_call(
        body,
        grid=(grid,),
        in_specs=[
            pl.BlockSpec((N_CONT, B), lambda i: (0, 0)),
            pl.BlockSpec((N_CONT, CB), lambda i: (0, i)),
            pl.BlockSpec((N_CONT, 1), lambda i: (0, 0)),
            pl.BlockSpec((N_CONT, 1), lambda i: (0, 0)),
            pl.BlockSpec((N_CONT, D, 1), lambda i: (0, 0, 0)),
            pl.BlockSpec((N_CONT, D, 1), lambda i: (0, 0, 0)),
        ],
        out_specs=pl.BlockSpec((N_CONT, D, CB), lambda i: (0, 0, i)),
        out_shape=jax.ShapeDtypeStruct((N_CONT, D, B), jnp.float32),
        compiler_params=pltpu.CompilerParams(
            dimension_semantics=("arbitrary",),
        ),
    )(x_t, x_t, gamma, beta, w_t, b_t)


def kernel(x, categorical, cont_embed_weight, cont_embed_bias, bn_gamma, bn_beta, cat_tables):
    tab_t = cat_tables.transpose(0, 2, 1)
    cat_t = categorical.T
    x_t = x.T
    gamma = bn_gamma.reshape(N_CONT, 1)
    beta = bn_beta.reshape(N_CONT, 1)
    w_t = cont_embed_weight.reshape(N_CONT, D, 1)
    b_t = cont_embed_bias.reshape(N_CONT, D, 1)

    tab_lin = _tc_repack(tab_t).reshape(N_CAT * PLANE)

    xcat_t = _sc_gather(cat_t, tab_lin)

    xcont_t = _tc_cont(x_t, gamma, beta, w_t, b_t)

    out_t = jnp.concatenate([xcont_t, xcat_t], axis=0)
    return out_t.transpose(2, 0, 1)


# speedup vs baseline: 16.2341x; 1.1012x over previous
import functools

import jax
import jax.numpy as jnp
import numpy as np
from jax import lax
from jax.experimental import pallas as pl
from jax.experimental.pallas import tpu as pltpu
from jax.experimental.pallas import tpu_sc as plsc

B = 16384
N_CONT = 13
N_CAT = 26
V = 100001
D = 16
EPS = 1e-5

V_PAD = 100352
PLANE = D * V_PAD

NC = 2
NS = 16
NW = NC * NS
ROWS_PER_W = B // NW
BCHUNK = 512
N_BCHUNK = ROWS_PER_W // BCHUNK

CB = 2048


def _tc_repack(tab_t):
    def body(t_ref, o_ref):
        o_ref[...] = t_ref[0].reshape(D, V_PAD // 128, 128)

    return pl.pallas_call(
        body,
        grid=(N_CAT,),
        in_specs=[pl.BlockSpec((1, D, V_PAD), lambda i: (i, 0, 0))],
        out_specs=pl.BlockSpec((D, V_PAD // 128, 128), lambda i: (i, 0, 0)),
        out_shape=jax.ShapeDtypeStruct((N_CAT * D, V_PAD // 128, 128), jnp.float32),
        compiler_params=pltpu.CompilerParams(
            dimension_semantics=("arbitrary",),
        ),
    )(tab_t)


def _sc_gather(cat_t, tab_lin):
    mesh = plsc.VectorSubcoreMesh(
        core_axis_name="c", subcore_axis_name="s", num_cores=NC, num_subcores=NS
    )

    @functools.partial(
        pl.kernel,
        out_type=jax.ShapeDtypeStruct((N_CONT + N_CAT, D, B), jnp.float32),
        name="sc_embedding_gather",
        mesh=mesh,
        scratch_types=[
            pltpu.VMEM((2, BCHUNK), jnp.int32),
            pltpu.VMEM((2, D, BCHUNK), jnp.float32),
            pltpu.SemaphoreType.DMA,
            pltpu.SemaphoreType.DMA,
            pltpu.SemaphoreType.DMA,
            pltpu.SemaphoreType.DMA,
        ],
        compiler_params=pltpu.CompilerParams(use_tc_tiling_on_sc=False),
    )
    def body(cat_hbm, tab_hbm, out_hbm, idx_v, val_v, sem_i, sem_g0, sem_g1, sem_o):
        wid = lax.axis_index("s") * NC + lax.axis_index("c")
        base = wid * ROWS_PER_W

        def fire(t, par):
            def one(d, carry):
                off = pl.multiple_of((t * D + d) * V_PAD, 128)
                sem = [sem_g0, sem_g1][par]
                pltpu.async_copy(
                    tab_hbm.at[pl.ds(off, V_PAD)].at[idx_v.at[par]],
                    val_v.at[par, d],
                    sem,
                )
                return carry

            lax.fori_loop(0, D, one, 0)

        def drain_val(par):
            sem = [sem_g0, sem_g1][par]
            pltpu.make_async_copy(
                out_hbm.at[0, :, pl.ds(0, BCHUNK)], val_v.at[par], sem
            ).wait()

        def drain_out():
            pltpu.make_async_copy(
                val_v.at[0], out_hbm.at[0, :, pl.ds(0, BCHUNK)], sem_o
            ).wait()

        pltpu.sync_copy(cat_hbm.at[0, pl.ds(base, BCHUNK)], idx_v.at[0])
        fire(0, 0)

        def step(t, carry):
            par = lax.rem(t, 2)
            nxt = 1 - par

            @pl.when(t + 1 < N_CAT)
            def _():
                pltpu.async_copy(
                    cat_hbm.at[t + 1, pl.ds(base, BCHUNK)], idx_v.at[nxt], sem_i
                ).wait()

                @pl.when(t >= 1)
                def _():
                    drain_out()

                @pl.when(nxt == 0)
                def _():
                    fire(t + 1, 0)

                @pl.when(nxt == 1)
                def _():
                    fire(t + 1, 1)

            @pl.when(par == 0)
            def _():
                drain_val(0)
                pltpu.async_copy(
                    val_v.at[0], out_hbm.at[N_CONT + t, :, pl.ds(base, BCHUNK)],
                    sem_o,
                )

            @pl.when(par == 1)
            def _():
                drain_val(1)
                pltpu.async_copy(
                    val_v.at[1], out_hbm.at[N_CONT + t, :, pl.ds(base, BCHUNK)],
                    sem_o,
                )

            return carry

        lax.fori_loop(0, N_CAT, step, 0)
        drain_out()
        drain_out()

    return body(cat_t, tab_lin)


def _tc_cont(x_t, gamma, beta, w_t, b_t, scout):
    def body(x_ref, xc_ref, g_ref, be_ref, w_ref, bb_ref, sc_ref, o_ref):
        xv = x_ref[...]
        mean = jnp.mean(xv, axis=1, keepdims=True)
        var = jnp.mean(xv * xv, axis=1, keepdims=True) - mean * mean
        inv = lax.rsqrt(var + EPS) * g_ref[...]
        xc = (xc_ref[...] - mean) * inv + be_ref[...]
        o_ref[...] = (
            w_ref[...] * xc[:, None, :] + bb_ref[...]
        )

    grid = B // CB
    return pl.pallas_call(
        body,
        grid=(grid,),
        in_specs=[
            pl.BlockSpec((N_CONT, B), lambda i: (0, 0)),
            pl.BlockSpec((N_CONT, CB), lambda i: (0, i)),
            pl.BlockSpec((N_CONT, 1), lambda i: (0, 0)),
            pl.BlockSpec((N_CONT, 1), lambda i: (0, 0)),
            pl.BlockSpec((N_CONT, D, 1), lambda i: (0, 0, 0)),
            pl.BlockSpec((N_CONT, D, 1), lambda i: (0, 0, 0)),
            pl.BlockSpec(memory_space=pl.ANY),
        ],
        out_specs=pl.BlockSpec((N_CONT, D, CB), lambda i: (0, 0, i)),
        out_shape=jax.ShapeDtypeStruct((N_CONT + N_CAT, D, B), jnp.float32),
        input_output_aliases={6: 0},
        compiler_params=pltpu.CompilerParams(
            dimension_semantics=("arbitrary",),
        ),
    )(x_t, x_t, gamma, beta, w_t, b_t, scout)


def kernel(x, categorical, cont_embed_weight, cont_embed_bias, bn_gamma, bn_beta, cat_tables):
    tab_t = cat_tables.transpose(0, 2, 1)
    cat_t = categorical.T
    x_t = x.T
    gamma = bn_gamma.reshape(N_CONT, 1)
    beta = bn_beta.reshape(N_CONT, 1)
    w_t = cont_embed_weight.reshape(N_CONT, D, 1)
    b_t = cont_embed_bias.reshape(N_CONT, D, 1)

    tab_lin = _tc_repack(tab_t).reshape(N_CAT * PLANE)

    scout = _sc_gather(cat_t, tab_lin)

    out_t = _tc_cont(x_t, gamma, beta, w_t, b_t, scout)
    return out_t.transpose(2, 0, 1)


# speedup vs baseline: 16.2541x; 1.0012x over previous
import functools

import jax
import jax.numpy as jnp
import numpy as np
from jax import lax
from jax.experimental import pallas as pl
from jax.experimental.pallas import tpu as pltpu
from jax.experimental.pallas import tpu_sc as plsc

B = 16384
N_CONT = 13
N_CAT = 26
V = 100001
D = 16
EPS = 1e-5

V_PAD = 100352
PLANE = D * V_PAD

NC = 2
NS = 16
NW = NC * NS
ROWS_PER_W = B // NW
BCHUNK = 512
N_BCHUNK = ROWS_PER_W // BCHUNK

CB = 2048


def _tc_repack(tab_t):
    def body(t_ref, o_ref):
        o_ref[...] = t_ref[0].reshape(PLANE)

    return pl.pallas_call(
        body,
        grid=(N_CAT,),
        in_specs=[pl.BlockSpec((1, D, V_PAD), lambda i: (i, 0, 0))],
        out_specs=pl.BlockSpec((PLANE,), lambda i: (i,)),
        out_shape=jax.ShapeDtypeStruct((N_CAT * PLANE,), jnp.float32),
        compiler_params=pltpu.CompilerParams(
            dimension_semantics=("arbitrary",),
        ),
    )(tab_t)


def _sc_gather(cat_t, tab_lin):
    mesh = plsc.VectorSubcoreMesh(
        core_axis_name="c", subcore_axis_name="s", num_cores=NC, num_subcores=NS
    )

    @functools.partial(
        pl.kernel,
        out_type=jax.ShapeDtypeStruct((N_CONT + N_CAT, D, B), jnp.float32),
        name="sc_embedding_gather",
        mesh=mesh,
        scratch_types=[
            pltpu.VMEM((2, BCHUNK), jnp.int32),
            pltpu.VMEM((2, D, BCHUNK), jnp.float32),
            pltpu.SemaphoreType.DMA,
            pltpu.SemaphoreType.DMA,
            pltpu.SemaphoreType.DMA,
            pltpu.SemaphoreType.DMA,
        ],
        compiler_params=pltpu.CompilerParams(use_tc_tiling_on_sc=False),
    )
    def body(cat_hbm, tab_hbm, out_hbm, idx_v, val_v, sem_i, sem_g0, sem_g1, sem_o):
        wid = lax.axis_index("s") * NC + lax.axis_index("c")
        base = wid * ROWS_PER_W

        def fire(t, par):
            def one(d, carry):
                off = pl.multiple_of((t * D + d) * V_PAD, 128)
                sem = [sem_g0, sem_g1][par]
                pltpu.async_copy(
                    tab_hbm.at[pl.ds(off, V_PAD)].at[idx_v.at[par]],
                    val_v.at[par, d],
                    sem,
                )
                return carry

            lax.fori_loop(0, D, one, 0)

        def drain_val(par):
            sem = [sem_g0, sem_g1][par]
            pltpu.make_async_copy(
                out_hbm.at[0, :, pl.ds(0, BCHUNK)], val_v.at[par], sem
            ).wait()

        def drain_out():
            pltpu.make_async_copy(
                val_v.at[0], out_hbm.at[0, :, pl.ds(0, BCHUNK)], sem_o
            ).wait()

        pltpu.sync_copy(cat_hbm.at[0, pl.ds(base, BCHUNK)], idx_v.at[0])
        fire(0, 0)

        def step(t, carry):
            par = lax.rem(t, 2)
            nxt = 1 - par

            @pl.when(t + 1 < N_CAT)
            def _():
                pltpu.async_copy(
                    cat_hbm.at[t + 1, pl.ds(base, BCHUNK)], idx_v.at[nxt], sem_i
                ).wait()

                @pl.when(t >= 1)
                def _():
                    drain_out()

                @pl.when(nxt == 0)
                def _():
                    fire(t + 1, 0)

                @pl.when(nxt == 1)
                def _():
                    fire(t + 1, 1)

            @pl.when(par == 0)
            def _():
                drain_val(0)
                pltpu.async_copy(
                    val_v.at[0], out_hbm.at[N_CONT + t, :, pl.ds(base, BCHUNK)],
                    sem_o,
                )

            @pl.when(par == 1)
            def _():
                drain_val(1)
                pltpu.async_copy(
                    val_v.at[1], out_hbm.at[N_CONT + t, :, pl.ds(base, BCHUNK)],
                    sem_o,
                )

            return carry

        lax.fori_loop(0, N_CAT, step, 0)
        drain_out()
        drain_out()

    return body(cat_t, tab_lin)


def _tc_cont(x_t, gamma, beta, w_t, b_t, scout):
    def body(x_ref, xc_ref, g_ref, be_ref, w_ref, bb_ref, sc_ref, o_ref):
        xv = x_ref[...]
        mean = jnp.mean(xv, axis=1, keepdims=True)
        var = jnp.mean(xv * xv, axis=1, keepdims=True) - mean * mean
        inv = lax.rsqrt(var + EPS) * g_ref[...]
        xc = (xc_ref[...] - mean) * inv + be_ref[...]
        o_ref[...] = (
            w_ref[...] * xc[:, None, :] + bb_ref[...]
        )

    grid = B // CB
    return pl.pallas_call(
        body,
        grid=(grid,),
        in_specs=[
            pl.BlockSpec((N_CONT, B), lambda i: (0, 0)),
            pl.BlockSpec((N_CONT, CB), lambda i: (0, i)),
            pl.BlockSpec((N_CONT, 1), lambda i: (0, 0)),
            pl.BlockSpec((N_CONT, 1), lambda i: (0, 0)),
            pl.BlockSpec((N_CONT, D, 1), lambda i: (0, 0, 0)),
            pl.BlockSpec((N_CONT, D, 1), lambda i: (0, 0, 0)),
            pl.BlockSpec(memory_space=pl.ANY),
        ],
        out_specs=pl.BlockSpec((N_CONT, D, CB), lambda i: (0, 0, i)),
        out_shape=jax.ShapeDtypeStruct((N_CONT + N_CAT, D, B), jnp.float32),
        input_output_aliases={6: 0},
        compiler_params=pltpu.CompilerParams(
            dimension_semantics=("arbitrary",),
        ),
    )(x_t, x_t, gamma, beta, w_t, b_t, scout)


def kernel(x, categorical, cont_embed_weight, cont_embed_bias, bn_gamma, bn_beta, cat_tables):
    tab_t = cat_tables.transpose(0, 2, 1)
    cat_t = categorical.T
    x_t = x.T
    gamma = bn_gamma.reshape(N_CONT, 1)
    beta = bn_beta.reshape(N_CONT, 1)
    w_t = cont_embed_weight.reshape(N_CONT, D, 1)
    b_t = cont_embed_bias.reshape(N_CONT, D, 1)

    tab_lin = _tc_repack(tab_t)

    scout = _sc_gather(cat_t, tab_lin)

    out_t = _tc_cont(x_t, gamma, beta, w_t, b_t, scout)
    return out_t.transpose(2, 0, 1)
